# Initial kernel scaffold; baseline (speedup 1.0000x reference)
import jax
import jax.numpy as jnp
from jax.experimental import pallas as pl


def kernel(x, edge_index, edge_attr, Wl, bl, Wr, br, We, att, bias, W1, b1, W2, b2):
    raise NotImplementedError("write your pallas kernel here")


# speedup vs baseline: 2.3731x; 2.3731x over previous
import jax
import jax.numpy as jnp
from jax import lax
from jax.experimental import pallas as pl
from jax.experimental.pallas import tpu as pltpu
from jax.experimental.pallas import tpu_sc as plsc

N = 10000
E = 320000
D = 128
H = 4
C = 128
C2 = C // 2
C4 = C // 4
HC = H * C
HID = 256
OUT = 128

NC = 2
NS = 16
NW = NC * NS
L = 16

NP = 10240
NPT = NP // NS
AF = 7
ALEN = AF * N
B1 = 48
G1 = B1 // L
EP = 321024
EPT = EP // NW
EPS = E // NS
B = 80
G = B // L
BR = 1000
GRID = N // BR

_mesh = plsc.VectorSubcoreMesh(
    core_axis_name="c", subcore_axis_name="s", num_cores=NC, num_subcores=NS)
_sc_params = pltpu.CompilerParams(needs_layout_passes=False)


def _lin_body(x_ref, wl_ref, bl_ref, wr_ref, br_ref, *outs):
    x = x_ref[...]
    xl = jnp.dot(x, wl_ref[...], preferred_element_type=jnp.float32) + bl_ref[...]
    xr = jnp.dot(x, wr_ref[...], preferred_element_type=jnp.float32) + br_ref[...]
    for h in range(H):
        outs[h][...] = xl[:, h * C:(h + 1) * C]
        outs[H + h][...] = xr[:, h * C:(h + 1) * C]


def _stage_a(x, Wl, bl, Wr, br):
    return pl.pallas_call(
        _lin_body,
        grid=(GRID,),
        in_specs=[
            pl.BlockSpec((BR, D), lambda i: (i, 0)),
            pl.BlockSpec((D, HC), lambda i: (0, 0)),
            pl.BlockSpec((HC,), lambda i: (0,)),
            pl.BlockSpec((D, HC), lambda i: (0, 0)),
            pl.BlockSpec((HC,), lambda i: (0,)),
        ],
        out_specs=[pl.BlockSpec((BR, C), lambda i: (i, 0))] * (2 * H),
        out_shape=[jax.ShapeDtypeStruct((N, C), jnp.float32)] * (2 * H),
    )(x, Wl, bl, Wr, br)


def _sc1_body(src_hbm, dst_hbm, eaf_hbm,
              xl0, xl1, xl2, xl3, xr0, xr1, xr2, xr3,
              we_hbm, att_hbm,
              exr0, exr1, exr2, exr3, accum_hbm,
              src_v, dst_v, eaf_v,
              xlv0, xlv1, xlv2, xlv3, xrv0, xrv1, xrv2, xrv3,
              exb0, exb1, exb2, exb3, rec_v,
              we_v, att_v, acc_v, gsem):
    xl_hbm = [xl0, xl1, xl2, xl3]
    xr_hbm = [xr0, xr1, xr2, xr3]
    xl_v = [xlv0, xlv1, xlv2, xlv3]
    xr_v = [xrv0, xrv1, xrv2, xrv3]
    exb = [exb0, exb1, exb2, exb3]
    exr_hbm = [exr0, exr1, exr2, exr3]

    cid = lax.axis_index("c")
    sid = lax.axis_index("s")
    wid = cid * NS + sid

    pltpu.sync_copy(we_hbm, we_v)
    pltpu.sync_copy(att_hbm, att_v)

    iota = jnp.arange(L, dtype=jnp.int32)
    zeros16 = jnp.zeros((L,), jnp.float32)
    ones16 = jnp.ones((L,), jnp.float32)
    mask7 = iota < AF
    iotaN = iota * N

    def zr_body(i, _):
        acc_v[pl.ds(i * L, L)] = zeros16
        return 0
    lax.fori_loop(0, (ALEN + 16) // L, zr_body, 0)

    def batch_body(g, _):
        base = wid * EPT + g * B1
        pltpu.sync_copy(src_hbm.at[pl.ds(base, B1)], src_v)
        pltpu.sync_copy(dst_hbm.at[pl.ds(base, B1)], dst_v)
        pltpu.sync_copy(eaf_hbm.at[pl.ds(base * 2, 2 * B1)], eaf_v.at[pl.ds(0, 2 * B1)])
        cps = []
        for h in range(H):
            cps.append(pltpu.async_copy(xl_hbm[h].at[src_v], xl_v[h], gsem))
            cps.append(pltpu.async_copy(xr_hbm[h].at[dst_v], xr_v[h], gsem))
        for cp in cps:
            cp.wait()

        eas = []
        gms = []
        for grp in range(G1):
            eids = iota + grp * L
            c0 = jnp.zeros((L,), jnp.int32)
            ea0 = plsc.load_gather(eaf_v, [eids * 2])
            ea1 = plsc.load_gather(eaf_v, [eids * 2 + 1])
            gm = (jnp.full((L,), g * B1 + grp * L, jnp.int32) + iota
                  + jnp.full((L,), 0, jnp.int32) * 0) < (E - wid * EPT)
            eas.append((ea0, ea1))
            gms.append(gm)
            plsc.store_scatter(rec_v, [eids, c0 + 4],
                               jnp.where(gm, ones16, 0.0))
            plsc.store_scatter(rec_v, [eids, c0 + 5], ea0)
            plsc.store_scatter(rec_v, [eids, c0 + 6], ea1)

        for h in range(H):
            def jbody(j, accs, h=h):
                jf = jnp.full((L,), j, jnp.int32)
                kf = jnp.full((L,), h * C, jnp.int32) + jf
                w0 = plsc.load_gather(we_v, [kf])
                w1 = plsc.load_gather(we_v, [kf + HC])
                at = plsc.load_gather(att_v, [kf])
                outs = []
                for grp in range(G1):
                    eids = iota + grp * L
                    ea0, ea1 = eas[grp]
                    xlg = plsc.load_gather(xl_v[h], [eids, jf])
                    xrg = plsc.load_gather(xr_v[h], [eids, jf])
                    m = xlg + xrg + ea0 * w0 + ea1 * w1
                    m = jnp.maximum(m, 0.0) + 0.2 * jnp.minimum(m, 0.0)
                    outs.append(accs[grp] + m * at)
                return tuple(outs)
            accs = lax.fori_loop(0, C, jbody, (zeros16,) * G1, unroll=2)
            for grp in range(G1):
                eids = iota + grp * L
                ex = jnp.where(gms[grp], jnp.exp(accs[grp]), 0.0)
                exb[h][pl.ds(grp * L, L)] = ex
                plsc.store_scatter(rec_v, [eids, jnp.full((L,), h, jnp.int32)], ex)

        for h in range(H):
            pltpu.sync_copy(exb[h], exr_hbm[h].at[pl.ds(base, B1)])

        for grp in range(G1):
            for l in range(L):
                e = grp * L + l
                ef = jnp.full((L,), e, jnp.int32)
                dsp = plsc.load_gather(dst_v, [ef])
                vals = plsc.load_gather(rec_v, [ef, iota])
                vals = jnp.where(mask7, vals, 0.0)
                idx = jnp.where(mask7, iotaN + dsp, jnp.full((L,), ALEN, jnp.int32))
                plsc.addupdate_scatter(acc_v, [idx], vals, mask=mask7)
        return 0

    lax.fori_loop(0, EPT // B1, batch_body, 0)

    pltpu.sync_copy(acc_v.at[pl.ds(0, ALEN)], accum_hbm.at[pl.ds(wid * ALEN, ALEN)])


def _stage_sc1(src, dst, eaf, xls, xrs, we_flat, att_flat):
    fn = pl.kernel(
        _sc1_body,
        out_type=[jax.ShapeDtypeStruct((EP,), jnp.float32)] * H + [
            jax.ShapeDtypeStruct((NW * ALEN,), jnp.float32),
        ],
        mesh=_mesh,
        compiler_params=_sc_params,
        scratch_types=[
            pltpu.VMEM((B1,), jnp.int32),
            pltpu.VMEM((B1,), jnp.int32),
            pltpu.VMEM((128,), jnp.float32),
        ] + [pltpu.VMEM((B1, C), jnp.float32)] * (2 * H)
          + [pltpu.VMEM((B1,), jnp.float32)] * 4
          + [pltpu.VMEM((B1, 32), jnp.float32)] + [
            pltpu.VMEM((2 * HC,), jnp.float32),
            pltpu.VMEM((HC,), jnp.float32),
            pltpu.VMEM((ALEN + 16,), jnp.float32),
            pltpu.SemaphoreType.DMA,
        ],
    )
    return fn(src, dst, eaf, *xls, *xrs, we_flat, att_flat)


def _selfloop_body(xl0, xl1, xl2, xl3, xr0, xr1, xr2, xr3,
                   acc_ref, wer_ref, attr_ref,
                   invden_ref, *oinit):
    xl = [xl0[...], xl1[...], xl2[...], xl3[...]]
    xr = [xr0[...], xr1[...], xr2[...], xr3[...]]
    acc = jnp.sum(acc_ref[...], axis=2)
    cnt = jnp.clip(acc[:, 4], 1.0)
    la0 = acc[:, 5] / cnt
    la1 = acc[:, 6] / cnt
    rows = []
    for h in range(H):
        ev = la0[:, None] * wer_ref[0, h][None, :] + la1[:, None] * wer_ref[1, h][None, :]
        m = xl[h] + xr[h] + ev
        a = jnp.maximum(m, 0.0) + 0.2 * jnp.minimum(m, 0.0)
        alpha = jnp.sum(a * attr_ref[h][None, :], axis=1)
        ex = jnp.exp(alpha)
        den = acc[:, h] + ex
        inv = 1.0 / (den + 1e-16)
        rows.append(inv)
        oinit[h][...] = xl[h] * ex[:, None]
    invden_ref[...] = jnp.stack(rows, axis=1)


def _stage_b(xls, xrs, accum, wer, attr):
    return pl.pallas_call(
        _selfloop_body,
        grid=(GRID,),
        in_specs=[pl.BlockSpec((BR, C), lambda i: (i, 0))] * (2 * H) + [
            pl.BlockSpec((BR, AF, NW), lambda i: (i, 0, 0)),
            pl.BlockSpec((2, H, C), lambda i: (0, 0, 0)),
            pl.BlockSpec((H, C), lambda i: (0, 0)),
        ],
        out_specs=[pl.BlockSpec((BR, H), lambda i: (i, 0))] + [
            pl.BlockSpec((BR, C), lambda i: (i, 0))] * H,
        out_shape=[jax.ShapeDtypeStruct((N, H), jnp.float32)] + [
            jax.ShapeDtypeStruct((N, C), jnp.float32)] * H,
    )(*xls, *xrs, accum, wer, attr)


def _sc2_body(src_hbm, dst_hbm, exr0, exr1, exr2, exr3,
              xl0, xl1, xl2, xl3, z_hbm,
              out0, out1, out2, out3, out4, out5, out6, out7,
              out8, out9, out10, out11, out12, out13, out14, out15,
              src_v, dst_v, exh_v, xlh_v, msg_v, zmsg_v, fl_v, zidx_v,
              outh_sh, gsem):
    xl_hbm = [xl0, xl1, xl2, xl3]
    exr_hbm = [exr0, exr1, exr2, exr3]
    out_hbm = [out0, out1, out2, out3, out4, out5, out6, out7,
               out8, out9, out10, out11, out12, out13, out14, out15]

    cid = lax.axis_index("c")
    sid = lax.axis_index("s")
    iota = jnp.arange(L, dtype=jnp.int32)
    zeros16 = jnp.zeros((L,), jnp.float32)

    def zm_body(i, _):
        zmsg_v[i, pl.ds(0, L)] = zeros16
        zmsg_v[i, pl.ds(L, L)] = zeros16
        msg_v[i, pl.ds(0, L)] = zeros16
        msg_v[i, pl.ds(L, L)] = zeros16
        return 0
    lax.fori_loop(0, 128, zm_body, 0)
    dump = jnp.full((L,), NP - 1, jnp.int32)
    for q in range(G, 8):
        dst_v[pl.ds(q * L, L)] = dump

    def _fill_idx(ch):
        for q in range(8):
            zidx_v[pl.ds(q * L, L)] = iota + (sid * NPT + ch * 128 + q * L)

    for phase in range(8):
        k, a = phase // 4, phase % 4
        def zr2_body(ch, _):
            _fill_idx(ch)
            pltpu.sync_copy(outh_sh.at[zidx_v], fl_v)

            def ng_body(i, _):
                fl_v[i, pl.ds(0, L)] = -fl_v[i, pl.ds(0, L)]
                fl_v[i, pl.ds(L, L)] = -fl_v[i, pl.ds(L, L)]
                return 0
            lax.fori_loop(0, 128, ng_body, 0)
            pltpu.sync_copy(fl_v, outh_sh.at[zidx_v], add=True)
            return 0
        lax.fori_loop(0, NPT // 128, zr2_body, 0)
        plsc.subcore_barrier()

        for c in range(NC):
            h = c * 2 + k
            t = 4 * h + a

            @pl.when(cid == c)
            def _head_scan(h=h, a=a):
                def batch_body(g, _):
                    base = sid * EPS + g * B
                    pltpu.sync_copy(src_hbm.at[pl.ds(base, B)], src_v)
                    pltpu.sync_copy(dst_hbm.at[pl.ds(base, B)], dst_v.at[pl.ds(0, B)])
                    pltpu.sync_copy(exr_hbm[h].at[pl.ds(base, B)], exh_v)
                    pltpu.async_copy(xl_hbm[h].at[src_v], xlh_v, gsem).wait()

                    for grp in range(G):
                        eids = iota + grp * L
                        sl = pl.ds(grp * L, L)
                        w = exh_v[sl]

                        def jbody(j, _, eids=eids, w=w, a=a):
                            jf = jnp.full((L,), j, jnp.int32)
                            xlg = plsc.load_gather(
                                xlh_v, [eids, jf + jnp.full((L,), a * C4, jnp.int32)])
                            plsc.store_scatter(msg_v, [eids, jf], xlg * w)
                            return 0
                        lax.fori_loop(0, C4, jbody, 0, unroll=4)

                    pltpu.sync_copy(msg_v, outh_sh.at[dst_v], add=True)
                    return 0

                lax.fori_loop(0, EPS // B, batch_body, 0)

        plsc.subcore_barrier()
        for c in range(NC):
            h = c * 2 + k
            t = 4 * h + a

            @pl.when(cid == c)
            def _head_flush(t=t):
                def fl_body(ch, _):
                    _fill_idx(ch)
                    pltpu.sync_copy(outh_sh.at[zidx_v], fl_v)
                    pltpu.sync_copy(
                        fl_v, out_hbm[t].at[pl.ds(sid * NPT + ch * 128, 128)])
                    return 0
                lax.fori_loop(0, NPT // 128, fl_body, 0)
        plsc.subcore_barrier()


def _stage_sc2(src, dst, exrec, xls, zrows):
    fn = pl.kernel(
        _sc2_body,
        out_type=[jax.ShapeDtypeStruct((NP, C4), jnp.float32)] * (4 * H),
        mesh=_mesh,
        compiler_params=_sc_params,
        scratch_types=[
            pltpu.VMEM((B,), jnp.int32),
            pltpu.VMEM((128,), jnp.int32),
            pltpu.VMEM((B,), jnp.float32),
            pltpu.VMEM((B, C), jnp.float32),
            pltpu.VMEM((128, C4), jnp.float32),
            pltpu.VMEM((128, C4), jnp.float32),
            pltpu.VMEM((128, C4), jnp.float32),
            pltpu.VMEM((128,), jnp.int32),
            pltpu.VMEM_SHARED((NP, C4), jnp.float32),
            pltpu.SemaphoreType.DMA,
        ],
    )
    return fn(src, dst, *exrec, *xls, zrows)


def _mlp_body(o0, o1, o2, o3, o4, o5, o6, o7,
              o8, o9, o10, o11, o12, o13, o14, o15, s0, s1, s2, s3, inv_ref,
              biasr_ref, w1r_ref, b1_ref, w2_ref, b2_ref, out_ref):
    on = [o0, o1, o2, o3, o4, o5, o6, o7,
          o8, o9, o10, o11, o12, o13, o14, o15]
    sl = [s0, s1, s2, s3]
    acc = jnp.zeros((BR, HID), jnp.float32) + b1_ref[...]
    for h in range(H):
        shalf = sl[h][...]
        inv = inv_ref[:, h][:, None]
        for a in range(4):
            t = ((on[4 * h + a][...] + shalf[:, a * C4:(a + 1) * C4]) * inv
                 + biasr_ref[h][a * C4:(a + 1) * C4][None, :])
            acc = acc + jnp.dot(t, w1r_ref[h][a * C4:(a + 1) * C4, :],
                                preferred_element_type=jnp.float32)
    hid = jnp.maximum(acc, 0.0)
    out_ref[...] = jnp.dot(hid, w2_ref[...], preferred_element_type=jnp.float32) + b2_ref[...]


def _stage_c(outn, oinit, invden, biasr, w1r, b1, w2, b2):
    return pl.pallas_call(
        _mlp_body,
        grid=(GRID,),
        in_specs=[pl.BlockSpec((BR, C4), lambda i: (i, 0))] * (4 * H)
                 + [pl.BlockSpec((BR, C), lambda i: (i, 0))] * H + [
            pl.BlockSpec((BR, H), lambda i: (i, 0)),
            pl.BlockSpec((H, C), lambda i: (0, 0)),
            pl.BlockSpec((H, C, HID), lambda i: (0, 0, 0)),
            pl.BlockSpec((HID,), lambda i: (0,)),
            pl.BlockSpec((HID, OUT), lambda i: (0, 0)),
            pl.BlockSpec((OUT,), lambda i: (0,)),
        ],
        out_specs=pl.BlockSpec((BR, OUT), lambda i: (i, 0)),
        out_shape=jax.ShapeDtypeStruct((N, OUT), jnp.float32),
    )(*outn, *oinit, invden, biasr, w1r, b1, w2, b2)


def kernel(x, edge_index, edge_attr, Wl, bl, Wr, br, We, att, bias, W1, b1, W2, b2):
    src = edge_index[0]
    dst = edge_index[1]
    eaf = edge_attr.reshape(-1)
    we_flat = We.reshape(-1)
    att_flat = att.reshape(-1)
    wer = We.reshape(2, H, C)
    attr = att.reshape(H, C)
    biasr = bias.reshape(H, C)
    w1r = W1.reshape(H, C, HID)
    zhalf = jnp.zeros((NP, C4), jnp.float32)
    npad = EP - E
    srcp = jnp.concatenate([src, jnp.zeros((npad,), jnp.int32)])
    dstp = jnp.concatenate([dst, jnp.zeros((npad,), jnp.int32)])
    eafp = jnp.concatenate([eaf, jnp.zeros((2 * npad,), jnp.float32)])

    lin = _stage_a(x, Wl, bl, Wr, br)
    xls, xrs = lin[:H], lin[H:]

    sc1 = _stage_sc1(srcp, dstp, eafp, xls, xrs, we_flat, att_flat)
    exrec = [r[:E] for r in sc1[:H]]
    accum = sc1[H]

    acc_t = jnp.transpose(accum.reshape(NW, AF, N), (2, 1, 0))
    bres = _stage_b(xls, xrs, acc_t, wer, attr)
    invden, oinit = bres[0], bres[1:]

    outn = _stage_sc2(src, dst, exrec, xls, zhalf)

    return _stage_c([o[:N] for o in outn], oinit, invden, biasr, w1r, b1, W2, b2)
